# baseline (speedup 1.0000x reference)
import functools

import jax
import jax.numpy as jnp
from jax import lax
from jax.experimental import pallas as pl
from jax.experimental.pallas import tpu as pltpu
from jax.experimental.pallas import tpu_sc as plsc

NUM_HEADS = 8
NUM_POINTS = 4


def _prep_body(q_ref, rpt_ref, wax_ref, bax_ref, wox_ref, box_ref,
               woy_ref, boy_ref, aw_ref, w0_ref, w1_ref, w2_ref, w3_ref,
               i0_ref, i1_ref, i2_ref, i3_ref, *, hgrid, wgrid, h, p, nsub):
    q = q_ref[0]
    cdim = (((0,), (1,)), ((), ()))

    logit = lax.dot_general(wax_ref[...], q, cdim,
                            preferred_element_type=jnp.float32)
    logit = logit + jnp.transpose(bax_ref[...], (1, 0))
    for hh in range(h):
        sub = logit[hh * p:(hh + 1) * p]
        m = jnp.max(sub, axis=0, keepdims=True)
        e = jnp.exp(sub - m)
        aw = e / jnp.sum(e, axis=0, keepdims=True)
        aw_ref[0, hh] = aw.reshape(2 * p, -1)

    offx = lax.dot_general(wox_ref[...], q, cdim,
                           preferred_element_type=jnp.float32)
    offx = offx + jnp.transpose(box_ref[...], (1, 0))
    offy = lax.dot_general(woy_ref[...], q, cdim,
                           preferred_element_type=jnp.float32)
    offy = offy + jnp.transpose(boy_ref[...], (1, 0))
    rpt = rpt_ref[0]
    gx = rpt[0:1] * 2.0 - 1.0 + offx
    gy = rpt[1:2] * 2.0 - 1.0 + offy
    ix = (gx + 1.0) * (wgrid / 2.0) - 0.5
    iy = (gy + 1.0) * (hgrid / 2.0) - 0.5
    x0 = jnp.floor(ix)
    y0 = jnp.floor(iy)
    wx1 = ix - x0
    wx0 = 1.0 - wx1
    wy1 = iy - y0
    wy0 = 1.0 - wy1

    wrefs = (w0_ref, w1_ref, w2_ref, w3_ref)
    irefs = (i0_ref, i1_ref, i2_ref, i3_ref)
    for k, (dy, dx, wyc, wxc) in enumerate(((0, 0, wy0, wx0), (0, 1, wy0, wx1),
                                            (1, 0, wy1, wx0), (1, 1, wy1, wx1))):
        xc = x0 + dx
        yc = y0 + dy
        valid = ((xc >= 0.0) & (xc <= wgrid - 1.0)
                 & (yc >= 0.0) & (yc <= hgrid - 1.0))
        xi = jnp.clip(xc, 0.0, wgrid - 1.0).astype(jnp.int32)
        yi = jnp.clip(yc, 0.0, hgrid - 1.0).astype(jnp.int32)
        wk = jnp.where(valid, wyc * wxc, 0.0)
        ik = yi * int(wgrid) + xi
        for hh in range(h):
            wrefs[k][0, hh] = wk[:, hh * nsub:(hh + 1) * nsub]
            irefs[k][0, hh] = ik[:, hh * nsub:(hh + 1) * nsub]


def _sc_body(value_ref, aw_ref, w0_ref, w1_ref, w2_ref, w3_ref,
             i0_ref, i1_ref, i2_ref, i3_ref, out_ref,
             table_v, awb_v, wb0, wb1, wb2, wb3, ib0, ib1, ib2, ib3, obuf_v,
             *, num_bh, h, nc, nw, ngroups, hd, hw):
    wid = lax.axis_index("s") * nc + lax.axis_index("c")
    lane = lax.iota(jnp.int32, 16)
    lane4 = lane * 4
    wbufs = (wb0, wb1, wb2, wb3)
    ibufs = (ib0, ib1, ib2, ib3)

    ntasks = (num_bh + nw - 1) // nw
    for t in range(ntasks):
        bh = wid + t * nw

        @pl.when(bh < num_bh)
        def _():
            b = bh // h
            hh = bh % h
            pltpu.sync_copy(value_ref.at[bh], table_v)
            pltpu.sync_copy(aw_ref.at[b, hh], awb_v)
            for src, dst in zip((w0_ref, w1_ref, w2_ref, w3_ref), wbufs):
                pltpu.sync_copy(src.at[b, hh], dst)
            for src, dst in zip((i0_ref, i1_ref, i2_ref, i3_ref), ibufs):
                pltpu.sync_copy(src.at[b, hh], dst)

            @plsc.parallel_loop(0, ngroups)
            def group(g):
                hd2 = hd // 2
                colv = lane + g * 16
                for half in range(2):
                    acc = [jnp.zeros((16,), jnp.float32) for _ in range(hd2)]
                    for pp in range(4):
                        s = lane4 + (g * 64 + pp)
                        srow = jnp.bitwise_and(s, 31)
                        scol = lax.shift_right_logical(s, 5)
                        af = lane + (pp * 1024 + g * 16)
                        arow = lax.shift_right_logical(af, 9)
                        acol = jnp.bitwise_and(af, 511)
                        av = plsc.load_gather(awb_v, [arow, acol])
                        wv = [plsc.load_gather(wbufs[k], [srow, scol]) * av
                              for k in range(4)]
                        iv = [plsc.load_gather(ibufs[k], [srow, scol])
                              for k in range(4)]
                        for ci in range(hd2):
                            c = half * hd2 + ci
                            tsl = table_v.at[pl.ds(c * hw, hw)]
                            for k in range(4):
                                gv = plsc.load_gather(tsl, [iv[k]])
                                acc[ci] = acc[ci] + gv * wv[k]
                    for ci in range(hd2):
                        c = half * hd2 + ci
                        plsc.store_scatter(obuf_v, [jnp.full((16,), c, jnp.int32),
                                                    colv], acc[ci])

            pltpu.sync_copy(obuf_v, out_ref.at[bh])


def _out_body(sf_ref, wo_ref, bo_ref, out_ref):
    out_ref[0] = lax.dot_general(sf_ref[0], wo_ref[...],
                                 (((0,), (0,)), ((), ())),
                                 preferred_element_type=jnp.float32) + bo_ref[...]


def kernel(query, reference_points, value, W_off, b_off, W_attn, b_attn,
           W_out, b_out, value_spatial_shapes):
    B, N, C = query.shape
    H, W = value.shape[2], value.shape[3]
    h, p = NUM_HEADS, NUM_POINTS
    hd = C // h
    hp = h * p
    nsub = N // h
    shape_dep = (value_spatial_shapes[0] - H) + (value_spatial_shapes[1] - W)

    w_off_r = W_off.reshape(C, h, p, 2)
    w_off_x = w_off_r[..., 0].reshape(C, hp)
    w_off_y = w_off_r[..., 1].reshape(C, hp)
    b_off_r = b_off.reshape(h, p, 2)
    b_off_x = b_off_r[..., 0].reshape(1, hp)
    b_off_y = b_off_r[..., 1].reshape(1, hp)
    b_attn_r = b_attn.reshape(1, hp)
    rpt = jnp.transpose(reference_points, (0, 2, 1))

    num_bh = B * h

    plan = pl.pallas_call(
        functools.partial(_prep_body, hgrid=float(H), wgrid=float(W),
                          h=h, p=p, nsub=nsub),
        grid=(B,),
        in_specs=[
            pl.BlockSpec((1, N, C), lambda b: (b, 0, 0)),
            pl.BlockSpec((1, 2, N), lambda b: (b, 0, 0)),
            pl.BlockSpec((C, hp), lambda b: (0, 0)),
            pl.BlockSpec((1, hp), lambda b: (0, 0)),
            pl.BlockSpec((C, hp), lambda b: (0, 0)),
            pl.BlockSpec((1, hp), lambda b: (0, 0)),
            pl.BlockSpec((C, hp), lambda b: (0, 0)),
            pl.BlockSpec((1, hp), lambda b: (0, 0)),
        ],
        out_specs=[pl.BlockSpec((1, h, 2 * p, N // 2), lambda b: (b, 0, 0, 0))] +
                  [pl.BlockSpec((1, h, hp, nsub), lambda b: (b, 0, 0, 0))] * 8,
        out_shape=[jax.ShapeDtypeStruct((B, h, 2 * p, N // 2), jnp.float32)] +
                  [jax.ShapeDtypeStruct((B, h, hp, nsub), jnp.float32)] * 4 +
                  [jax.ShapeDtypeStruct((B, h, hp, nsub), jnp.int32)] * 4,
    )(query, rpt, W_attn, b_attn_r, w_off_x, b_off_x, w_off_y, b_off_y)
    aw_t, w0, w1, w2, w3, i0, i1, i2, i3 = plan

    value_flat = value.reshape(num_bh, hd * H * W)

    info = plsc.get_sparse_core_info()
    nw = info.num_cores * info.num_subcores
    ngroups = N // 16

    sc_fn = functools.partial(
        pl.kernel,
        mesh=plsc.VectorSubcoreMesh(core_axis_name="c", subcore_axis_name="s"),
        compiler_params=pltpu.CompilerParams(needs_layout_passes=False),
        out_type=jax.ShapeDtypeStruct((num_bh, hd, N), jnp.float32),
        scratch_types=[
            pltpu.VMEM((hd * H * W,), jnp.float32),
            pltpu.VMEM((2 * p, N // 2), jnp.float32),
        ] + [pltpu.VMEM((hp, nsub), jnp.float32)] * 4
          + [pltpu.VMEM((hp, nsub), jnp.int32)] * 4
          + [pltpu.VMEM((hd, N), jnp.float32)],
    )(functools.partial(_sc_body, num_bh=num_bh, h=h, nc=info.num_cores, nw=nw,
                        ngroups=ngroups, hd=hd, hw=H * W))
    sf = sc_fn(value_flat, aw_t, w0, w1, w2, w3, i0, i1, i2, i3)

    sf3 = sf.reshape(B, C, N)
    b_out2 = (b_out + jnp.float32(shape_dep)).reshape(1, C)
    out = pl.pallas_call(
        _out_body,
        grid=(B,),
        in_specs=[
            pl.BlockSpec((1, C, N), lambda b: (b, 0, 0)),
            pl.BlockSpec((C, C), lambda b: (0, 0)),
            pl.BlockSpec((1, C), lambda b: (0, 0)),
        ],
        out_specs=pl.BlockSpec((1, N, C), lambda b: (b, 0, 0)),
        out_shape=jax.ShapeDtypeStruct((B, N, C), jnp.float32),
    )(sf3, W_out, b_out2)
    return out


# speedup vs baseline: 1.2604x; 1.2604x over previous
import functools

import jax
import jax.numpy as jnp
from jax import lax
from jax.experimental import pallas as pl
from jax.experimental.pallas import tpu as pltpu
from jax.experimental.pallas import tpu_sc as plsc

NUM_HEADS = 8
NUM_POINTS = 4


def _prep_body(q_ref, rpt_ref, wax_ref, bax_ref, wox_ref, box_ref,
               woy_ref, boy_ref, aw_ref, w0_ref, w1_ref, w2_ref, w3_ref,
               i0_ref, i1_ref, i2_ref, i3_ref, *, hgrid, wgrid, h, p, nsub):
    q = q_ref[0]
    cdim = (((0,), (1,)), ((), ()))

    logit = lax.dot_general(wax_ref[...], q, cdim,
                            preferred_element_type=jnp.float32)
    logit = logit + jnp.transpose(bax_ref[...], (1, 0))
    for hh in range(h):
        sub = logit[hh * p:(hh + 1) * p]
        m = jnp.max(sub, axis=0, keepdims=True)
        e = jnp.exp(sub - m)
        aw = e / jnp.sum(e, axis=0, keepdims=True)
        aw_ref[0, hh] = aw.reshape(2 * p, -1)

    offx = lax.dot_general(wox_ref[...], q, cdim,
                           preferred_element_type=jnp.float32)
    offx = offx + jnp.transpose(box_ref[...], (1, 0))
    offy = lax.dot_general(woy_ref[...], q, cdim,
                           preferred_element_type=jnp.float32)
    offy = offy + jnp.transpose(boy_ref[...], (1, 0))
    rpt = rpt_ref[0]
    gx = rpt[0:1] * 2.0 - 1.0 + offx
    gy = rpt[1:2] * 2.0 - 1.0 + offy
    ix = (gx + 1.0) * (wgrid / 2.0) - 0.5
    iy = (gy + 1.0) * (hgrid / 2.0) - 0.5
    x0 = jnp.floor(ix)
    y0 = jnp.floor(iy)
    wx1 = ix - x0
    wx0 = 1.0 - wx1
    wy1 = iy - y0
    wy0 = 1.0 - wy1

    wrefs = (w0_ref, w1_ref, w2_ref, w3_ref)
    irefs = (i0_ref, i1_ref, i2_ref, i3_ref)
    for k, (dy, dx, wyc, wxc) in enumerate(((0, 0, wy0, wx0), (0, 1, wy0, wx1),
                                            (1, 0, wy1, wx0), (1, 1, wy1, wx1))):
        xc = x0 + dx
        yc = y0 + dy
        valid = ((xc >= 0.0) & (xc <= wgrid - 1.0)
                 & (yc >= 0.0) & (yc <= hgrid - 1.0))
        xi = jnp.clip(xc, 0.0, wgrid - 1.0).astype(jnp.int32)
        yi = jnp.clip(yc, 0.0, hgrid - 1.0).astype(jnp.int32)
        wk = jnp.where(valid, wyc * wxc, 0.0)
        ik = yi * int(wgrid) + xi
        for hh in range(h):
            wrefs[k][0, hh] = wk[:, hh * nsub:(hh + 1) * nsub]
            irefs[k][0, hh] = ik[:, hh * nsub:(hh + 1) * nsub]


def _sc_body(value_ref, aw_ref, w0_ref, w1_ref, w2_ref, w3_ref,
             i0_ref, i1_ref, i2_ref, i3_ref, out_ref,
             table_v, awb_v, wb0, wb1, wb2, wb3, ib0, ib1, ib2, ib3, obuf_v,
             *, num_bh, h, nc, nw, ngroups, hd, hw):
    wid = lax.axis_index("s") * nc + lax.axis_index("c")
    lane = lax.iota(jnp.int32, 16)
    lane4 = lane * 4
    wbufs = (wb0, wb1, wb2, wb3)
    ibufs = (ib0, ib1, ib2, ib3)

    ntasks = (num_bh + nw - 1) // nw
    for t in range(ntasks):
        bh = wid + t * nw

        @pl.when(bh < num_bh)
        def _():
            b = bh // h
            hh = bh % h
            pltpu.sync_copy(value_ref.at[bh], table_v)
            pltpu.sync_copy(aw_ref.at[b, hh], awb_v)
            for src, dst in zip((w0_ref, w1_ref, w2_ref, w3_ref), wbufs):
                pltpu.sync_copy(src.at[b, hh], dst)
            for src, dst in zip((i0_ref, i1_ref, i2_ref, i3_ref), ibufs):
                pltpu.sync_copy(src.at[b, hh], dst)

            @plsc.parallel_loop(0, ngroups)
            def group(g):
                hd2 = hd // 2
                colv = lane + g * 16
                for half in range(2):
                    acc = [jnp.zeros((16,), jnp.float32) for _ in range(hd2)]
                    for pp in range(4):
                        s = lane4 + (g * 64 + pp)
                        srow = jnp.bitwise_and(s, 31)
                        scol = lax.shift_right_logical(s, 5)
                        af = lane + (pp * 1024 + g * 16)
                        arow = lax.shift_right_logical(af, 9)
                        acol = jnp.bitwise_and(af, 511)
                        av = plsc.load_gather(awb_v, [arow, acol])
                        wv = [plsc.load_gather(wbufs[k], [srow, scol]) * av
                              for k in range(4)]
                        iv = [plsc.load_gather(ibufs[k], [srow, scol])
                              for k in range(4)]
                        for ci in range(hd2):
                            c = half * hd2 + ci
                            for k in range(4):
                                gv = plsc.load_gather(table_v, [iv[k] + c * hw])
                                acc[ci] = acc[ci] + gv * wv[k]
                    for ci in range(hd2):
                        c = half * hd2 + ci
                        plsc.store_scatter(obuf_v, [jnp.full((16,), c, jnp.int32),
                                                    colv], acc[ci])

            pltpu.sync_copy(obuf_v, out_ref.at[bh])


def _out_body(sf_ref, wo_ref, bo_ref, out_ref):
    out_ref[0] = lax.dot_general(sf_ref[0], wo_ref[...],
                                 (((0,), (0,)), ((), ())),
                                 preferred_element_type=jnp.float32) + bo_ref[...]


def kernel(query, reference_points, value, W_off, b_off, W_attn, b_attn,
           W_out, b_out, value_spatial_shapes):
    B, N, C = query.shape
    H, W = value.shape[2], value.shape[3]
    h, p = NUM_HEADS, NUM_POINTS
    hd = C // h
    hp = h * p
    nsub = N // h
    shape_dep = (value_spatial_shapes[0] - H) + (value_spatial_shapes[1] - W)

    w_off_r = W_off.reshape(C, h, p, 2)
    w_off_x = w_off_r[..., 0].reshape(C, hp)
    w_off_y = w_off_r[..., 1].reshape(C, hp)
    b_off_r = b_off.reshape(h, p, 2)
    b_off_x = b_off_r[..., 0].reshape(1, hp)
    b_off_y = b_off_r[..., 1].reshape(1, hp)
    b_attn_r = b_attn.reshape(1, hp)
    rpt = jnp.transpose(reference_points, (0, 2, 1))

    num_bh = B * h

    plan = pl.pallas_call(
        functools.partial(_prep_body, hgrid=float(H), wgrid=float(W),
                          h=h, p=p, nsub=nsub),
        grid=(B,),
        in_specs=[
            pl.BlockSpec((1, N, C), lambda b: (b, 0, 0)),
            pl.BlockSpec((1, 2, N), lambda b: (b, 0, 0)),
            pl.BlockSpec((C, hp), lambda b: (0, 0)),
            pl.BlockSpec((1, hp), lambda b: (0, 0)),
            pl.BlockSpec((C, hp), lambda b: (0, 0)),
            pl.BlockSpec((1, hp), lambda b: (0, 0)),
            pl.BlockSpec((C, hp), lambda b: (0, 0)),
            pl.BlockSpec((1, hp), lambda b: (0, 0)),
        ],
        out_specs=[pl.BlockSpec((1, h, 2 * p, N // 2), lambda b: (b, 0, 0, 0))] +
                  [pl.BlockSpec((1, h, hp, nsub), lambda b: (b, 0, 0, 0))] * 8,
        out_shape=[jax.ShapeDtypeStruct((B, h, 2 * p, N // 2), jnp.float32)] +
                  [jax.ShapeDtypeStruct((B, h, hp, nsub), jnp.float32)] * 4 +
                  [jax.ShapeDtypeStruct((B, h, hp, nsub), jnp.int32)] * 4,
    )(query, rpt, W_attn, b_attn_r, w_off_x, b_off_x, w_off_y, b_off_y)
    aw_t, w0, w1, w2, w3, i0, i1, i2, i3 = plan

    value_flat = value.reshape(num_bh, hd * H * W)

    info = plsc.get_sparse_core_info()
    nw = info.num_cores * info.num_subcores
    ngroups = N // 16

    sc_fn = functools.partial(
        pl.kernel,
        mesh=plsc.VectorSubcoreMesh(core_axis_name="c", subcore_axis_name="s"),
        compiler_params=pltpu.CompilerParams(needs_layout_passes=False),
        out_type=jax.ShapeDtypeStruct((num_bh, hd, N), jnp.float32),
        scratch_types=[
            pltpu.VMEM((hd * H * W,), jnp.float32),
            pltpu.VMEM((2 * p, N // 2), jnp.float32),
        ] + [pltpu.VMEM((hp, nsub), jnp.float32)] * 4
          + [pltpu.VMEM((hp, nsub), jnp.int32)] * 4
          + [pltpu.VMEM((hd, N), jnp.float32)],
    )(functools.partial(_sc_body, num_bh=num_bh, h=h, nc=info.num_cores, nw=nw,
                        ngroups=ngroups, hd=hd, hw=H * W))
    sf = sc_fn(value_flat, aw_t, w0, w1, w2, w3, i0, i1, i2, i3)

    sf3 = sf.reshape(B, C, N)
    b_out2 = (b_out + jnp.float32(shape_dep)).reshape(1, C)
    out = pl.pallas_call(
        _out_body,
        grid=(B,),
        in_specs=[
            pl.BlockSpec((1, C, N), lambda b: (b, 0, 0)),
            pl.BlockSpec((C, C), lambda b: (0, 0)),
            pl.BlockSpec((1, C), lambda b: (0, 0)),
        ],
        out_specs=pl.BlockSpec((1, N, C), lambda b: (b, 0, 0)),
        out_shape=jax.ShapeDtypeStruct((B, N, C), jnp.float32),
    )(sf3, W_out, b_out2)
    return out
